# Initial kernel scaffold; baseline (speedup 1.0000x reference)
import jax
import jax.numpy as jnp
from jax.experimental import pallas as pl


def kernel(predict, groundth, keep_num):
    raise NotImplementedError("write your pallas kernel here")


# speedup vs baseline: 6.6182x; 6.6182x over previous
import functools

import jax
import jax.numpy as jnp
from jax import lax
from jax.experimental import pallas as pl
from jax.experimental.pallas import tpu as pltpu
from jax.experimental.pallas import tpu_sc as plsc

N = 32 * 32768
NW = 32
PER_W = N // NW
NV = PER_W // 16
L1B = 1024
L2B = 2048
L3B = 1024
KEEP = N // 2
RATE = 0.5

_mesh = plsc.VectorSubcoreMesh(core_axis_name="c", subcore_axis_name="s")


def _wid():
    return lax.axis_index("s") * 2 + lax.axis_index("c")


@functools.partial(
    pl.kernel,
    mesh=_mesh,
    compiler_params=pltpu.CompilerParams(needs_layout_passes=False),
    out_type=[
        jax.ShapeDtypeStruct((N,), jnp.float32),
        jax.ShapeDtypeStruct((NW, 16 * L1B), jnp.int32),
        jax.ShapeDtypeStruct((NW, 16 * L1B), jnp.float32),
    ],
    scratch_types=[
        pltpu.VMEM((PER_W,), jnp.float32),
        pltpu.VMEM((PER_W,), jnp.float32),
        pltpu.VMEM((16 * L1B,), jnp.int32),
        pltpu.VMEM((16 * L1B,), jnp.float32),
    ],
)
def _sc_pass1(p_hbm, g_hbm, loss_hbm, cnt_hbm, sum_hbm, pbuf, gbuf, cnt, sm):
    wid = _wid()
    base = wid * PER_W
    pltpu.sync_copy(p_hbm.at[pl.ds(base, PER_W)], pbuf)
    pltpu.sync_copy(g_hbm.at[pl.ds(base, PER_W)], gbuf)

    zi = jnp.zeros((16,), jnp.int32)
    zf = jnp.zeros((16,), jnp.float32)

    def zero_body(i, carry):
        cnt[pl.ds(i * 16, 16)] = zi
        sm[pl.ds(i * 16, 16)] = zf
        return carry

    lax.fori_loop(0, L1B, zero_body, 0)

    lane = lax.iota(jnp.int32, 16) * L1B
    ones = jnp.ones((16,), jnp.int32)

    def body(i, carry):
        off = i * 16
        p = pbuf[pl.ds(off, 16)]
        g = gbuf[pl.ds(off, 16)]
        loss = jnp.abs(jnp.where(g > 0.0, p, 0.0) - g)
        pbuf[pl.ds(off, 16)] = loss
        bits = plsc.bitcast(loss, jnp.int32)
        idx = lane + (bits >> 21)
        plsc.addupdate_scatter(cnt, [idx], ones)
        plsc.addupdate_scatter(sm, [idx], loss)
        return carry

    lax.fori_loop(0, NV, body, 0)

    pltpu.sync_copy(pbuf, loss_hbm.at[pl.ds(base, PER_W)])
    pltpu.sync_copy(cnt, cnt_hbm.at[wid])
    pltpu.sync_copy(sm, sum_hbm.at[wid])


@functools.partial(
    pl.kernel,
    mesh=_mesh,
    compiler_params=pltpu.CompilerParams(needs_layout_passes=False),
    out_type=[
        jax.ShapeDtypeStruct((NW, 16 * L2B), jnp.int32),
        jax.ShapeDtypeStruct((NW, 16 * L2B), jnp.float32),
    ],
    scratch_types=[
        pltpu.VMEM((PER_W,), jnp.float32),
        pltpu.VMEM((16,), jnp.int32),
        pltpu.VMEM((16 * L2B,), jnp.int32),
        pltpu.VMEM((16 * L2B,), jnp.float32),
    ],
)
def _sc_pass2(loss_hbm, p1_hbm, cnt_hbm, sum_hbm, lbuf, pv, cnt, sm):
    wid = _wid()
    base = wid * PER_W
    pltpu.sync_copy(loss_hbm.at[pl.ds(base, PER_W)], lbuf)
    pltpu.sync_copy(p1_hbm.at[0], pv)
    b1 = pv[...]

    zi = jnp.zeros((16,), jnp.int32)
    zf = jnp.zeros((16,), jnp.float32)

    def zero_body(i, carry):
        cnt[pl.ds(i * 16, 16)] = zi
        sm[pl.ds(i * 16, 16)] = zf
        return carry

    lax.fori_loop(0, L2B, zero_body, 0)

    lane = lax.iota(jnp.int32, 16) * L2B
    ones = jnp.ones((16,), jnp.int32)

    def body(i, carry):
        loss = lbuf[pl.ds(i * 16, 16)]
        bits = plsc.bitcast(loss, jnp.int32)
        act = (bits >> 21) == b1
        idx = lane + ((bits >> 10) & (L2B - 1))
        plsc.addupdate_scatter(cnt, [idx], ones, mask=act)
        plsc.addupdate_scatter(sm, [idx], loss, mask=act)
        return carry

    lax.fori_loop(0, NV, body, 0)

    pltpu.sync_copy(cnt, cnt_hbm.at[wid])
    pltpu.sync_copy(sm, sum_hbm.at[wid])


@functools.partial(
    pl.kernel,
    mesh=_mesh,
    compiler_params=pltpu.CompilerParams(needs_layout_passes=False),
    out_type=[
        jax.ShapeDtypeStruct((NW, 16 * L3B), jnp.int32),
    ],
    scratch_types=[
        pltpu.VMEM((PER_W,), jnp.float32),
        pltpu.VMEM((16,), jnp.int32),
        pltpu.VMEM((16 * L3B,), jnp.int32),
    ],
)
def _sc_pass3(loss_hbm, p2_hbm, cnt_hbm, lbuf, pv, cnt):
    wid = _wid()
    base = wid * PER_W
    pltpu.sync_copy(loss_hbm.at[pl.ds(base, PER_W)], lbuf)
    pltpu.sync_copy(p2_hbm.at[0], pv)
    pref = pv[...]

    zi = jnp.zeros((16,), jnp.int32)

    def zero_body(i, carry):
        cnt[pl.ds(i * 16, 16)] = zi
        return carry

    lax.fori_loop(0, L3B, zero_body, 0)

    lane = lax.iota(jnp.int32, 16) * L3B
    ones = jnp.ones((16,), jnp.int32)

    def body(i, carry):
        loss = lbuf[pl.ds(i * 16, 16)]
        bits = plsc.bitcast(loss, jnp.int32)
        act = (bits >> 10) == pref
        idx = lane + (bits & (L3B - 1))
        plsc.addupdate_scatter(cnt, [idx], ones, mask=act)
        return carry

    lax.fori_loop(0, NV, body, 0)

    pltpu.sync_copy(cnt, cnt_hbm.at[wid])


def _suffix_ge(c, nb):
    iota = lax.broadcasted_iota(jnp.int32, (1, nb), 1)
    t = c
    s = 1
    while s < nb:
        r = pltpu.roll(t, nb - s, axis=1)
        t = t + jnp.where(iota < nb - s, r, 0)
        s *= 2
    return t


def _tc_decide1(cnt, sm):
    def body(cnt_ref, sum_ref, p1_ref, kv_ref, sa_ref):
        c = jnp.sum(cnt_ref[...], axis=0, keepdims=True)
        s = jnp.sum(sum_ref[...], axis=0, keepdims=True)
        t = _suffix_ge(c, L1B)
        iota = lax.broadcasted_iota(jnp.int32, (1, L1B), 1)
        mask = t >= KEEP
        b1 = jnp.max(jnp.where(mask, iota, 0))
        above = iota > b1
        cnt_above = jnp.sum(jnp.where(above, c, 0))
        sum_above = jnp.sum(jnp.where(above, s, 0.0))
        k1 = KEEP - cnt_above
        p1_ref[...] = jnp.full((1, 16), b1, jnp.int32)
        kv_ref[...] = jnp.full((1, 128), k1, jnp.int32)
        sa_ref[...] = jnp.full((1, 128), sum_above, jnp.float32)

    return pl.pallas_call(
        body,
        out_shape=[
            jax.ShapeDtypeStruct((1, 16), jnp.int32),
            jax.ShapeDtypeStruct((1, 128), jnp.int32),
            jax.ShapeDtypeStruct((1, 128), jnp.float32),
        ],
    )(cnt, sm)


def _tc_decide2(cnt, sm, p1, kv, sa):
    def body(cnt_ref, sum_ref, p1_ref, kv_ref, sa_ref, p2_ref, kv2_ref, sa2_ref):
        b1 = jnp.max(p1_ref[...])
        k1 = jnp.max(kv_ref[...])
        sacc = jnp.max(sa_ref[...])
        c = jnp.sum(cnt_ref[...], axis=0, keepdims=True)
        s = jnp.sum(sum_ref[...], axis=0, keepdims=True)
        t = _suffix_ge(c, L2B)
        iota = lax.broadcasted_iota(jnp.int32, (1, L2B), 1)
        mask = t >= k1
        b2 = jnp.max(jnp.where(mask, iota, 0))
        above = iota > b2
        cnt_above = jnp.sum(jnp.where(above, c, 0))
        sum_above = jnp.sum(jnp.where(above, s, 0.0))
        k2 = k1 - cnt_above
        prefix = (b1 << 11) | b2
        p2_ref[...] = jnp.full((1, 16), prefix, jnp.int32)
        kv2_ref[...] = jnp.full((1, 128), k2, jnp.int32)
        sa2_ref[...] = jnp.full((1, 128), sacc + sum_above, jnp.float32)

    return pl.pallas_call(
        body,
        out_shape=[
            jax.ShapeDtypeStruct((1, 16), jnp.int32),
            jax.ShapeDtypeStruct((1, 128), jnp.int32),
            jax.ShapeDtypeStruct((1, 128), jnp.float32),
        ],
    )(cnt, sm, p1, kv, sa)


def _tc_decide3(cnt, p2, kv, sa):
    def body(cnt_ref, p2_ref, kv_ref, sa_ref, out_ref):
        prefix = jnp.max(p2_ref[...])
        k2 = jnp.max(kv_ref[...])
        sacc = jnp.max(sa_ref[...])
        c = jnp.sum(cnt_ref[...], axis=0, keepdims=True)
        cf = c.astype(jnp.float32)
        iota = lax.broadcasted_iota(jnp.int32, (1, L3B), 1)
        vals = lax.bitcast_convert_type((prefix << 10) | iota, jnp.float32)
        t = _suffix_ge(c, L3B)
        mask = t >= k2
        b3 = jnp.max(jnp.where(mask, iota, 0))
        above = iota > b3
        cnt_above = jnp.sum(jnp.where(above, c, 0))
        sum_above = jnp.sum(jnp.where(above, cf * vals, 0.0))
        kfin = (k2 - cnt_above).astype(jnp.float32)
        tval = jnp.sum(jnp.where(iota == b3, vals, 0.0))
        out_ref[...] = jnp.full((1, 128), sacc + sum_above + kfin * tval,
                                jnp.float32)

    return pl.pallas_call(
        body,
        out_shape=jax.ShapeDtypeStruct((1, 128), jnp.float32),
    )(cnt, p2, kv, sa)


def kernel(predict, groundth, keep_num):
    p = predict.reshape(-1)
    g = groundth.reshape(-1)
    loss, c1, s1 = _sc_pass1(p, g)
    p1, k1, sa1 = _tc_decide1(c1.reshape(NW * 16, L1B), s1.reshape(NW * 16, L1B))
    c2, s2 = _sc_pass2(loss, p1)
    p2, k2, sa2 = _tc_decide2(c2.reshape(NW * 16, L2B), s2.reshape(NW * 16, L2B),
                              p1, k1, sa1)
    (c3,) = _sc_pass3(loss, p2)
    total = _tc_decide3(c3.reshape(NW * 16, L3B), p2, k2, sa2)
    return total[0, 0] / (RATE * keep_num)


# speedup vs baseline: 10.6976x; 1.6164x over previous
import functools

import jax
import jax.numpy as jnp
from jax import lax
from jax.experimental import pallas as pl
from jax.experimental.pallas import tpu as pltpu
from jax.experimental.pallas import tpu_sc as plsc

N = 32 * 32768
NW = 32
PER_W = N // NW
NV = PER_W // 16
L1B = 1024
L2B = 2048
L3B = 1024
KEEP = N // 2
RATE = 0.5

_mesh = plsc.VectorSubcoreMesh(core_axis_name="c", subcore_axis_name="s")


def _wid():
    return lax.axis_index("s") * 2 + lax.axis_index("c")


@functools.partial(
    pl.kernel,
    mesh=_mesh,
    compiler_params=pltpu.CompilerParams(needs_layout_passes=False),
    out_type=[
        jax.ShapeDtypeStruct((N,), jnp.float32),
        jax.ShapeDtypeStruct((NW, 16 * L1B), jnp.int32),
        jax.ShapeDtypeStruct((NW, 16 * L1B), jnp.float32),
    ],
    scratch_types=[
        pltpu.VMEM((PER_W,), jnp.float32),
        pltpu.VMEM((PER_W,), jnp.float32),
        pltpu.VMEM((16 * L1B,), jnp.int32),
        pltpu.VMEM((16 * L1B,), jnp.float32),
    ],
)
def _sc_pass1(p_hbm, g_hbm, loss_hbm, cnt_hbm, sum_hbm, pbuf, gbuf, cnt, sm):
    wid = _wid()
    base = wid * PER_W
    pltpu.sync_copy(p_hbm.at[pl.ds(base, PER_W)], pbuf)
    pltpu.sync_copy(g_hbm.at[pl.ds(base, PER_W)], gbuf)

    zi = jnp.zeros((16,), jnp.int32)
    zf = jnp.zeros((16,), jnp.float32)

    @plsc.parallel_loop(0, 16 * L1B, 16, unroll=8)
    def _(off):
        cnt[pl.ds(off, 16)] = zi
        sm[pl.ds(off, 16)] = zf

    lane = lax.iota(jnp.int32, 16) * L1B
    ones = jnp.ones((16,), jnp.int32)

    @plsc.parallel_loop(0, PER_W, 16, unroll=8)
    def _(off):
        p = pbuf[pl.ds(off, 16)]
        g = gbuf[pl.ds(off, 16)]
        loss = jnp.abs(jnp.where(g > 0.0, p, 0.0) - g)
        pbuf[pl.ds(off, 16)] = loss
        bits = plsc.bitcast(loss, jnp.int32)
        idx = lane + (bits >> 21)
        plsc.addupdate_scatter(cnt, [idx], ones)
        plsc.addupdate_scatter(sm, [idx], loss)

    pltpu.sync_copy(pbuf, loss_hbm.at[pl.ds(base, PER_W)])
    pltpu.sync_copy(cnt, cnt_hbm.at[wid])
    pltpu.sync_copy(sm, sum_hbm.at[wid])


@functools.partial(
    pl.kernel,
    mesh=_mesh,
    compiler_params=pltpu.CompilerParams(needs_layout_passes=False),
    out_type=[
        jax.ShapeDtypeStruct((NW, 16 * L2B), jnp.int32),
        jax.ShapeDtypeStruct((NW, 16 * L2B), jnp.float32),
    ],
    scratch_types=[
        pltpu.VMEM((PER_W,), jnp.float32),
        pltpu.VMEM((16,), jnp.int32),
        pltpu.VMEM((16 * L2B,), jnp.int32),
        pltpu.VMEM((16 * L2B,), jnp.float32),
    ],
)
def _sc_pass2(loss_hbm, p1_hbm, cnt_hbm, sum_hbm, lbuf, pv, cnt, sm):
    wid = _wid()
    base = wid * PER_W
    pltpu.sync_copy(loss_hbm.at[pl.ds(base, PER_W)], lbuf)
    pltpu.sync_copy(p1_hbm.at[0], pv)
    b1 = pv[...]

    zi = jnp.zeros((16,), jnp.int32)
    zf = jnp.zeros((16,), jnp.float32)

    @plsc.parallel_loop(0, 16 * L2B, 16, unroll=8)
    def _(off):
        cnt[pl.ds(off, 16)] = zi
        sm[pl.ds(off, 16)] = zf

    lane = lax.iota(jnp.int32, 16) * L2B
    ones = jnp.ones((16,), jnp.int32)

    @plsc.parallel_loop(0, PER_W, 16, unroll=8)
    def _(off):
        loss = lbuf[pl.ds(off, 16)]
        bits = plsc.bitcast(loss, jnp.int32)
        act = (bits >> 21) == b1
        idx = lane + ((bits >> 10) & (L2B - 1))
        plsc.addupdate_scatter(cnt, [idx], ones, mask=act)
        plsc.addupdate_scatter(sm, [idx], loss, mask=act)

    pltpu.sync_copy(cnt, cnt_hbm.at[wid])
    pltpu.sync_copy(sm, sum_hbm.at[wid])


@functools.partial(
    pl.kernel,
    mesh=_mesh,
    compiler_params=pltpu.CompilerParams(needs_layout_passes=False),
    out_type=[
        jax.ShapeDtypeStruct((NW, 16 * L3B), jnp.int32),
    ],
    scratch_types=[
        pltpu.VMEM((PER_W,), jnp.float32),
        pltpu.VMEM((16,), jnp.int32),
        pltpu.VMEM((16 * L3B,), jnp.int32),
    ],
)
def _sc_pass3(loss_hbm, p2_hbm, cnt_hbm, lbuf, pv, cnt):
    wid = _wid()
    base = wid * PER_W
    pltpu.sync_copy(loss_hbm.at[pl.ds(base, PER_W)], lbuf)
    pltpu.sync_copy(p2_hbm.at[0], pv)
    pref = pv[...]

    zi = jnp.zeros((16,), jnp.int32)

    @plsc.parallel_loop(0, 16 * L3B, 16, unroll=8)
    def _(off):
        cnt[pl.ds(off, 16)] = zi

    lane = lax.iota(jnp.int32, 16) * L3B
    ones = jnp.ones((16,), jnp.int32)

    @plsc.parallel_loop(0, PER_W, 16, unroll=8)
    def _(off):
        loss = lbuf[pl.ds(off, 16)]
        bits = plsc.bitcast(loss, jnp.int32)
        act = (bits >> 10) == pref
        idx = lane + (bits & (L3B - 1))
        plsc.addupdate_scatter(cnt, [idx], ones, mask=act)

    pltpu.sync_copy(cnt, cnt_hbm.at[wid])


def _suffix_ge(c, nb):
    iota = lax.broadcasted_iota(jnp.int32, (1, nb), 1)
    t = c
    s = 1
    while s < nb:
        r = pltpu.roll(t, nb - s, axis=1)
        t = t + jnp.where(iota < nb - s, r, 0)
        s *= 2
    return t


def _tc_decide1(cnt, sm):
    def body(cnt_ref, sum_ref, p1_ref, kv_ref, sa_ref):
        c = jnp.sum(cnt_ref[...], axis=0, keepdims=True)
        s = jnp.sum(sum_ref[...], axis=0, keepdims=True)
        t = _suffix_ge(c, L1B)
        iota = lax.broadcasted_iota(jnp.int32, (1, L1B), 1)
        mask = t >= KEEP
        b1 = jnp.max(jnp.where(mask, iota, 0))
        above = iota > b1
        cnt_above = jnp.sum(jnp.where(above, c, 0))
        sum_above = jnp.sum(jnp.where(above, s, 0.0))
        k1 = KEEP - cnt_above
        p1_ref[...] = jnp.full((1, 16), b1, jnp.int32)
        kv_ref[...] = jnp.full((1, 128), k1, jnp.int32)
        sa_ref[...] = jnp.full((1, 128), sum_above, jnp.float32)

    return pl.pallas_call(
        body,
        out_shape=[
            jax.ShapeDtypeStruct((1, 16), jnp.int32),
            jax.ShapeDtypeStruct((1, 128), jnp.int32),
            jax.ShapeDtypeStruct((1, 128), jnp.float32),
        ],
    )(cnt, sm)


def _tc_decide2(cnt, sm, p1, kv, sa):
    def body(cnt_ref, sum_ref, p1_ref, kv_ref, sa_ref, p2_ref, kv2_ref, sa2_ref):
        b1 = jnp.max(p1_ref[...])
        k1 = jnp.max(kv_ref[...])
        sacc = jnp.max(sa_ref[...])
        c = jnp.sum(cnt_ref[...], axis=0, keepdims=True)
        s = jnp.sum(sum_ref[...], axis=0, keepdims=True)
        t = _suffix_ge(c, L2B)
        iota = lax.broadcasted_iota(jnp.int32, (1, L2B), 1)
        mask = t >= k1
        b2 = jnp.max(jnp.where(mask, iota, 0))
        above = iota > b2
        cnt_above = jnp.sum(jnp.where(above, c, 0))
        sum_above = jnp.sum(jnp.where(above, s, 0.0))
        k2 = k1 - cnt_above
        prefix = (b1 << 11) | b2
        p2_ref[...] = jnp.full((1, 16), prefix, jnp.int32)
        kv2_ref[...] = jnp.full((1, 128), k2, jnp.int32)
        sa2_ref[...] = jnp.full((1, 128), sacc + sum_above, jnp.float32)

    return pl.pallas_call(
        body,
        out_shape=[
            jax.ShapeDtypeStruct((1, 16), jnp.int32),
            jax.ShapeDtypeStruct((1, 128), jnp.int32),
            jax.ShapeDtypeStruct((1, 128), jnp.float32),
        ],
    )(cnt, sm, p1, kv, sa)


def _tc_decide3(cnt, p2, kv, sa):
    def body(cnt_ref, p2_ref, kv_ref, sa_ref, out_ref):
        prefix = jnp.max(p2_ref[...])
        k2 = jnp.max(kv_ref[...])
        sacc = jnp.max(sa_ref[...])
        c = jnp.sum(cnt_ref[...], axis=0, keepdims=True)
        cf = c.astype(jnp.float32)
        iota = lax.broadcasted_iota(jnp.int32, (1, L3B), 1)
        vals = lax.bitcast_convert_type((prefix << 10) | iota, jnp.float32)
        t = _suffix_ge(c, L3B)
        mask = t >= k2
        b3 = jnp.max(jnp.where(mask, iota, 0))
        above = iota > b3
        cnt_above = jnp.sum(jnp.where(above, c, 0))
        sum_above = jnp.sum(jnp.where(above, cf * vals, 0.0))
        kfin = (k2 - cnt_above).astype(jnp.float32)
        tval = jnp.sum(jnp.where(iota == b3, vals, 0.0))
        out_ref[...] = jnp.full((1, 128), sacc + sum_above + kfin * tval,
                                jnp.float32)

    return pl.pallas_call(
        body,
        out_shape=jax.ShapeDtypeStruct((1, 128), jnp.float32),
    )(cnt, p2, kv, sa)


def kernel(predict, groundth, keep_num):
    p = predict.reshape(-1)
    g = groundth.reshape(-1)
    loss, c1, s1 = _sc_pass1(p, g)
    p1, k1, sa1 = _tc_decide1(c1.reshape(NW * 16, L1B), s1.reshape(NW * 16, L1B))
    c2, s2 = _sc_pass2(loss, p1)
    p2, k2, sa2 = _tc_decide2(c2.reshape(NW * 16, L2B), s2.reshape(NW * 16, L2B),
                              p1, k1, sa1)
    (c3,) = _sc_pass3(loss, p2)
    total = _tc_decide3(c3.reshape(NW * 16, L3B), p2, k2, sa2)
    return total[0, 0] / (RATE * keep_num)


# speedup vs baseline: 14.9430x; 1.3969x over previous
import functools

import jax
import jax.numpy as jnp
from jax import lax
from jax.experimental import pallas as pl
from jax.experimental.pallas import tpu as pltpu
from jax.experimental.pallas import tpu_sc as plsc

N = 32 * 32768
NW = 32
PER_W = N // NW
L1B = 1024
L2B = 2048
L3B = 1024
KEEP = N // 2
RATE = 0.5

_mesh = plsc.VectorSubcoreMesh(core_axis_name="c", subcore_axis_name="s")


def _wid():
    return lax.axis_index("s") * 2 + lax.axis_index("c")


@functools.partial(
    pl.kernel,
    mesh=_mesh,
    compiler_params=pltpu.CompilerParams(needs_layout_passes=False),
    out_type=[
        jax.ShapeDtypeStruct((NW, PER_W), jnp.float32),
        jax.ShapeDtypeStruct((NW, L1B), jnp.int32),
        jax.ShapeDtypeStruct((NW, L1B), jnp.float32),
    ],
    scratch_types=[
        pltpu.VMEM((PER_W,), jnp.float32),
        pltpu.VMEM((PER_W,), jnp.float32),
        pltpu.VMEM((16 * L1B,), jnp.int32),
        pltpu.VMEM((16 * L1B,), jnp.float32),
        pltpu.VMEM((L1B,), jnp.int32),
        pltpu.VMEM((L1B,), jnp.float32),
    ],
)
def _sc_pass1(p_hbm, g_hbm, loss_hbm, cnt_hbm, sum_hbm, pbuf, gbuf, cnt, sm,
              cred, sred):
    wid = _wid()
    pltpu.sync_copy(p_hbm.at[wid], pbuf)
    pltpu.sync_copy(g_hbm.at[wid], gbuf)

    zi = jnp.zeros((16,), jnp.int32)
    zf = jnp.zeros((16,), jnp.float32)

    @plsc.parallel_loop(0, 16 * L1B, 16, unroll=8)
    def _(off):
        cnt[pl.ds(off, 16)] = zi
        sm[pl.ds(off, 16)] = zf

    lane = lax.iota(jnp.int32, 16) * L1B
    ones = jnp.ones((16,), jnp.int32)

    @plsc.parallel_loop(0, PER_W, 16, unroll=8)
    def _(off):
        p = pbuf[pl.ds(off, 16)]
        g = gbuf[pl.ds(off, 16)]
        loss = jnp.abs(jnp.where(g > 0.0, p, 0.0) - g)
        pbuf[pl.ds(off, 16)] = loss
        bits = plsc.bitcast(loss, jnp.int32)
        idx = lane + (bits >> 21)
        plsc.addupdate_scatter(cnt, [idx], ones)
        plsc.addupdate_scatter(sm, [idx], loss)

    @plsc.parallel_loop(0, L1B, 16, unroll=4)
    def _(g):
        acc_i = cnt[pl.ds(g, 16)]
        acc_f = sm[pl.ds(g, 16)]
        for l in range(1, 16):
            acc_i = acc_i + cnt[pl.ds(l * L1B + g, 16)]
            acc_f = acc_f + sm[pl.ds(l * L1B + g, 16)]
        cred[pl.ds(g, 16)] = acc_i
        sred[pl.ds(g, 16)] = acc_f

    pltpu.sync_copy(pbuf, loss_hbm.at[wid])
    pltpu.sync_copy(cred, cnt_hbm.at[wid])
    pltpu.sync_copy(sred, sum_hbm.at[wid])


@functools.partial(
    pl.kernel,
    mesh=_mesh,
    compiler_params=pltpu.CompilerParams(needs_layout_passes=False),
    out_type=[
        jax.ShapeDtypeStruct((NW, L2B), jnp.int32),
        jax.ShapeDtypeStruct((NW, L2B), jnp.float32),
    ],
    scratch_types=[
        pltpu.VMEM((PER_W,), jnp.float32),
        pltpu.VMEM((16,), jnp.int32),
        pltpu.VMEM((16 * L2B,), jnp.int32),
        pltpu.VMEM((16 * L2B,), jnp.float32),
        pltpu.VMEM((L2B,), jnp.int32),
        pltpu.VMEM((L2B,), jnp.float32),
    ],
)
def _sc_pass2(loss_hbm, p1_hbm, cnt_hbm, sum_hbm, lbuf, pv, cnt, sm, cred,
              sred):
    wid = _wid()
    pltpu.sync_copy(loss_hbm.at[wid], lbuf)
    pltpu.sync_copy(p1_hbm.at[0], pv)
    b1 = pv[...]

    zi = jnp.zeros((16,), jnp.int32)
    zf = jnp.zeros((16,), jnp.float32)

    @plsc.parallel_loop(0, 16 * L2B, 16, unroll=8)
    def _(off):
        cnt[pl.ds(off, 16)] = zi
        sm[pl.ds(off, 16)] = zf

    lane = lax.iota(jnp.int32, 16) * L2B
    ones = jnp.ones((16,), jnp.int32)

    @plsc.parallel_loop(0, PER_W, 16, unroll=8)
    def _(off):
        loss = lbuf[pl.ds(off, 16)]
        bits = plsc.bitcast(loss, jnp.int32)
        act = (bits >> 21) == b1
        idx = lane + ((bits >> 10) & (L2B - 1))
        plsc.addupdate_scatter(cnt, [idx], ones, mask=act)
        plsc.addupdate_scatter(sm, [idx], loss, mask=act)

    @plsc.parallel_loop(0, L2B, 16, unroll=4)
    def _(g):
        acc_i = cnt[pl.ds(g, 16)]
        acc_f = sm[pl.ds(g, 16)]
        for l in range(1, 16):
            acc_i = acc_i + cnt[pl.ds(l * L2B + g, 16)]
            acc_f = acc_f + sm[pl.ds(l * L2B + g, 16)]
        cred[pl.ds(g, 16)] = acc_i
        sred[pl.ds(g, 16)] = acc_f

    pltpu.sync_copy(cred, cnt_hbm.at[wid])
    pltpu.sync_copy(sred, sum_hbm.at[wid])


@functools.partial(
    pl.kernel,
    mesh=_mesh,
    compiler_params=pltpu.CompilerParams(needs_layout_passes=False),
    out_type=[
        jax.ShapeDtypeStruct((NW, L3B), jnp.int32),
    ],
    scratch_types=[
        pltpu.VMEM((PER_W,), jnp.float32),
        pltpu.VMEM((16,), jnp.int32),
        pltpu.VMEM((16 * L3B,), jnp.int32),
        pltpu.VMEM((L3B,), jnp.int32),
    ],
)
def _sc_pass3(loss_hbm, p2_hbm, cnt_hbm, lbuf, pv, cnt, cred):
    wid = _wid()
    pltpu.sync_copy(loss_hbm.at[wid], lbuf)
    pltpu.sync_copy(p2_hbm.at[0], pv)
    pref = pv[...]

    zi = jnp.zeros((16,), jnp.int32)

    @plsc.parallel_loop(0, 16 * L3B, 16, unroll=8)
    def _(off):
        cnt[pl.ds(off, 16)] = zi

    lane = lax.iota(jnp.int32, 16) * L3B
    ones = jnp.ones((16,), jnp.int32)

    @plsc.parallel_loop(0, PER_W, 16, unroll=8)
    def _(off):
        loss = lbuf[pl.ds(off, 16)]
        bits = plsc.bitcast(loss, jnp.int32)
        act = (bits >> 10) == pref
        idx = lane + (bits & (L3B - 1))
        plsc.addupdate_scatter(cnt, [idx], ones, mask=act)

    @plsc.parallel_loop(0, L3B, 16, unroll=4)
    def _(g):
        acc_i = cnt[pl.ds(g, 16)]
        for l in range(1, 16):
            acc_i = acc_i + cnt[pl.ds(l * L3B + g, 16)]
        cred[pl.ds(g, 16)] = acc_i

    pltpu.sync_copy(cred, cnt_hbm.at[wid])


def _suffix_ge(c, nb):
    iota = lax.broadcasted_iota(jnp.int32, (1, nb), 1)
    t = c
    s = 1
    while s < nb:
        r = pltpu.roll(t, nb - s, axis=1)
        t = t + jnp.where(iota < nb - s, r, 0)
        s *= 2
    return t


def _tc_decide1(cnt, sm):
    def body(cnt_ref, sum_ref, p1_ref, kv_ref, sa_ref):
        c = jnp.sum(cnt_ref[...], axis=0, keepdims=True)
        s = jnp.sum(sum_ref[...], axis=0, keepdims=True)
        t = _suffix_ge(c, L1B)
        iota = lax.broadcasted_iota(jnp.int32, (1, L1B), 1)
        mask = t >= KEEP
        b1 = jnp.max(jnp.where(mask, iota, 0))
        above = iota > b1
        cnt_above = jnp.sum(jnp.where(above, c, 0))
        sum_above = jnp.sum(jnp.where(above, s, 0.0))
        k1 = KEEP - cnt_above
        p1_ref[...] = jnp.full((1, 16), b1, jnp.int32)
        kv_ref[...] = jnp.full((1, 128), k1, jnp.int32)
        sa_ref[...] = jnp.full((1, 128), sum_above, jnp.float32)

    return pl.pallas_call(
        body,
        out_shape=[
            jax.ShapeDtypeStruct((1, 16), jnp.int32),
            jax.ShapeDtypeStruct((1, 128), jnp.int32),
            jax.ShapeDtypeStruct((1, 128), jnp.float32),
        ],
    )(cnt, sm)


def _tc_decide2(cnt, sm, p1, kv, sa):
    def body(cnt_ref, sum_ref, p1_ref, kv_ref, sa_ref, p2_ref, kv2_ref, sa2_ref):
        b1 = jnp.max(p1_ref[...])
        k1 = jnp.max(kv_ref[...])
        sacc = jnp.max(sa_ref[...])
        c = jnp.sum(cnt_ref[...], axis=0, keepdims=True)
        s = jnp.sum(sum_ref[...], axis=0, keepdims=True)
        t = _suffix_ge(c, L2B)
        iota = lax.broadcasted_iota(jnp.int32, (1, L2B), 1)
        mask = t >= k1
        b2 = jnp.max(jnp.where(mask, iota, 0))
        above = iota > b2
        cnt_above = jnp.sum(jnp.where(above, c, 0))
        sum_above = jnp.sum(jnp.where(above, s, 0.0))
        k2 = k1 - cnt_above
        prefix = (b1 << 11) | b2
        p2_ref[...] = jnp.full((1, 16), prefix, jnp.int32)
        kv2_ref[...] = jnp.full((1, 128), k2, jnp.int32)
        sa2_ref[...] = jnp.full((1, 128), sacc + sum_above, jnp.float32)

    return pl.pallas_call(
        body,
        out_shape=[
            jax.ShapeDtypeStruct((1, 16), jnp.int32),
            jax.ShapeDtypeStruct((1, 128), jnp.int32),
            jax.ShapeDtypeStruct((1, 128), jnp.float32),
        ],
    )(cnt, sm, p1, kv, sa)


def _tc_decide3(cnt, p2, kv, sa):
    def body(cnt_ref, p2_ref, kv_ref, sa_ref, out_ref):
        prefix = jnp.max(p2_ref[...])
        k2 = jnp.max(kv_ref[...])
        sacc = jnp.max(sa_ref[...])
        c = jnp.sum(cnt_ref[...], axis=0, keepdims=True)
        cf = c.astype(jnp.float32)
        iota = lax.broadcasted_iota(jnp.int32, (1, L3B), 1)
        vals = lax.bitcast_convert_type((prefix << 10) | iota, jnp.float32)
        t = _suffix_ge(c, L3B)
        mask = t >= k2
        b3 = jnp.max(jnp.where(mask, iota, 0))
        above = iota > b3
        cnt_above = jnp.sum(jnp.where(above, c, 0))
        sum_above = jnp.sum(jnp.where(above, cf * vals, 0.0))
        kfin = (k2 - cnt_above).astype(jnp.float32)
        tval = jnp.sum(jnp.where(iota == b3, vals, 0.0))
        out_ref[...] = jnp.full((1, 128), sacc + sum_above + kfin * tval,
                                jnp.float32)

    return pl.pallas_call(
        body,
        out_shape=jax.ShapeDtypeStruct((1, 128), jnp.float32),
    )(cnt, p2, kv, sa)


def kernel(predict, groundth, keep_num):
    loss, c1, s1 = _sc_pass1(predict, groundth)
    p1, k1, sa1 = _tc_decide1(c1, s1)
    c2, s2 = _sc_pass2(loss, p1)
    p2, k2, sa2 = _tc_decide2(c2, s2, p1, k1, sa1)
    (c3,) = _sc_pass3(loss, p2)
    total = _tc_decide3(c3, p2, k2, sa2)
    return total[0, 0] / (RATE * keep_num)
